# baseline (speedup 1.0000x reference)
import functools

import jax
import jax.numpy as jnp
from jax import lax
from jax.experimental import pallas as pl
from jax.experimental.pallas import tpu as pltpu
from jax.experimental.pallas import tpu_sc as plsc

EPS = 1e-5
NC = 2
NS = 16
NW = NC * NS
NPAD = 50176
D = 128


def _sc_gather(h, dst, src, chunk=200):
    E = dst.shape[0]
    per_w = E // NW
    n_chunks = per_w // chunk
    mesh = plsc.VectorSubcoreMesh(core_axis_name="c", subcore_axis_name="s")

    @functools.partial(
        pl.kernel, mesh=mesh,
        out_type=(jax.ShapeDtypeStruct((E, D), jnp.float32),
                  jax.ShapeDtypeStruct((E, D), jnp.float32)),
        scratch_types=[
            pltpu.VMEM((chunk,), jnp.int32),
            pltpu.VMEM((chunk,), jnp.int32),
            pltpu.VMEM((chunk, D), jnp.float32),
            pltpu.VMEM((chunk, D), jnp.float32),
            pltpu.SemaphoreType.DMA,
            pltpu.SemaphoreType.DMA,
        ],
    )
    def k(h_hbm, dst_hbm, src_hbm, xi_hbm, xj_hbm,
          idx_i, idx_j, rows_i, rows_j, sem_i, sem_j):
        wid = lax.axis_index("s") * NC + lax.axis_index("c")
        base_w = wid * per_w

        def body(ci, _):
            base = base_w + ci * chunk
            pltpu.sync_copy(dst_hbm.at[pl.ds(base, chunk)], idx_i)
            pltpu.sync_copy(src_hbm.at[pl.ds(base, chunk)], idx_j)
            cp_i = pltpu.async_copy(h_hbm.at[idx_i], rows_i, sem_i)
            cp_j = pltpu.async_copy(h_hbm.at[idx_j], rows_j, sem_j)
            cp_i.wait()
            cp_j.wait()
            pltpu.sync_copy(rows_i, xi_hbm.at[pl.ds(base, chunk)])
            pltpu.sync_copy(rows_j, xj_hbm.at[pl.ds(base, chunk)])
            return 0

        lax.fori_loop(0, n_chunks, body, 0)

    return k(h, dst, src)


def _sc_scatter(m, dstp, rowptr, chunk=512):
    FC = 16
    WIN = NPAD // NW
    zeros = jnp.zeros((WIN * FC + 16,), jnp.float32)
    mesh = plsc.VectorSubcoreMesh(core_axis_name="c", subcore_axis_name="s")

    @functools.partial(
        pl.kernel, mesh=mesh,
        compiler_params=pltpu.CompilerParams(needs_layout_passes=False),
        out_type=jax.ShapeDtypeStruct((NPAD * FC,), jnp.float32),
        scratch_types=[
            pltpu.VMEM((chunk,), jnp.int32),
            pltpu.VMEM((chunk, FC), jnp.float32),
            pltpu.VMEM((WIN * FC + 16,), jnp.float32),
            pltpu.VMEM((40,), jnp.int32),
        ],
    )
    def k(m_hbm, dst_hbm, rp_hbm, z_hbm, out_hbm, idx_v, vals_v, acc_v, rp_v):
        c = lax.axis_index("c")
        s = lax.axis_index("s")
        win = s * NC + c
        w0 = win * WIN
        pltpu.sync_copy(rp_hbm, rp_v)
        pltpu.sync_copy(z_hbm, acc_v)
        iota = lax.broadcasted_iota(jnp.int32, (16,), 0)

        def rp_read(i):
            grp = (i // 16) * 16
            vec = rp_v[pl.ds(grp, 16)]
            return jnp.sum(jnp.where(iota == i - grp, vec, 0))

        lo = rp_read(win)
        hi = rp_read(win + 1)
        lo_r = (lo // 8) * 8
        n_ch = (hi - lo_r + chunk - 1) // chunk

        def body(ci, _):
            base = lo_r + ci * chunk
            pltpu.sync_copy(dst_hbm.at[pl.ds(base, chunk)], idx_v)
            pltpu.sync_copy(m_hbm.at[pl.ds(base, chunk)], vals_v)

            def group(g, _):
                rows = g * 16 + iota
                idxv = idx_v[pl.ds(g * 16, 16)]
                ok = (idxv >= w0) & (idxv < w0 + WIN)
                a = jnp.where(ok, (idxv - w0) * FC, WIN * FC)
                for j in range(FC):
                    jf = jnp.full((16,), j, jnp.int32)
                    v = plsc.load_gather(vals_v, [rows, jf])
                    plsc.addupdate_scatter(acc_v, [a + j], v)
                return 0

            lax.fori_loop(0, chunk // 16, group, 0)
            return 0

        lax.fori_loop(0, n_ch, body, 0)
        pltpu.sync_copy(acc_v.at[pl.ds(0, WIN * FC)],
                        out_hbm.at[pl.ds(w0 * FC, WIN * FC)])

    return k(m, dstp, rowptr, zeros)


def _tc_pool(zr, batch_r, Nb=2000):
    N, F = zr.shape

    def kern(z_ref, b_ref, o_ref):
        brow = b_ref[...].reshape(1, Nb)
        rows = lax.broadcasted_iota(jnp.int32, (512, Nb), 0)
        oht = (rows == jnp.broadcast_to(brow, (512, Nb))).astype(jnp.float32)
        zcat = jnp.concatenate(
            [z_ref[...], jnp.ones((Nb, 128), jnp.float32)], axis=1)
        part = jnp.dot(oht, zcat, preferred_element_type=jnp.float32)

        @pl.when(pl.program_id(0) == 0)
        def _():
            o_ref[...] = part

        @pl.when(pl.program_id(0) > 0)
        def _():
            o_ref[...] += part

    return pl.pallas_call(
        kern,
        grid=(N // Nb,),
        in_specs=[
            pl.BlockSpec((Nb, F), lambda i: (i, 0)),
            pl.BlockSpec((1, 1, Nb), lambda i: (i, 0, 0)),
        ],
        out_specs=pl.BlockSpec((512, F + 128), lambda i: (0, 0)),
        out_shape=jax.ShapeDtypeStruct((512, F + 128), jnp.float32),
    )(zr, batch_r)


def _stats_block(y, dh):
    s = jnp.sum(y, axis=0, keepdims=True)
    q = jnp.sum(y * y, axis=0, keepdims=True)
    rows = lax.broadcasted_iota(jnp.int32, (8, dh), 0)
    return jnp.where(rows == 0, jnp.broadcast_to(s, (8, dh)),
                     jnp.where(rows == 1, jnp.broadcast_to(q, (8, dh)), 0.0))


def _tc_mm2_stats(Xi, Xj, Wa, Wb, b, Eb):
    E, d_i = Xi.shape
    d_j = Xj.shape[1]
    dh = Wa.shape[1]
    bb = jnp.broadcast_to(b[None, :], (8, dh))

    def kern(xi_ref, xj_ref, wa_ref, wb_ref, b_ref, y_ref, st_ref):
        y = jnp.dot(xi_ref[...], wa_ref[...],
                    preferred_element_type=jnp.float32)
        y = y + jnp.dot(xj_ref[...], wb_ref[...],
                        preferred_element_type=jnp.float32)
        y = y + b_ref[...][0:1, :]
        y_ref[...] = y
        st = _stats_block(y, dh)

        @pl.when(pl.program_id(0) == 0)
        def _():
            st_ref[...] = st

        @pl.when(pl.program_id(0) > 0)
        def _():
            st_ref[...] += st

    return pl.pallas_call(
        kern,
        grid=(E // Eb,),
        in_specs=[
            pl.BlockSpec((Eb, d_i), lambda i: (i, 0)),
            pl.BlockSpec((Eb, d_j), lambda i: (i, 0)),
            pl.BlockSpec((d_i, dh), lambda i: (0, 0)),
            pl.BlockSpec((d_j, dh), lambda i: (0, 0)),
            pl.BlockSpec((8, dh), lambda i: (0, 0)),
        ],
        out_specs=[
            pl.BlockSpec((Eb, dh), lambda i: (i, 0)),
            pl.BlockSpec((8, dh), lambda i: (0, 0)),
        ],
        out_shape=[
            jax.ShapeDtypeStruct((E, dh), jnp.float32),
            jax.ShapeDtypeStruct((8, dh), jnp.float32),
        ],
    )(Xi, Xj, Wa, Wb, bb)


def _tc_affine_mm_stats(Y1, sc, sh, W2, b2, Eb):
    E, dh_in = Y1.shape
    dh = W2.shape[1]
    scb = jnp.broadcast_to(sc[None, :], (8, dh_in))
    shb = jnp.broadcast_to(sh[None, :], (8, dh_in))
    bb = jnp.broadcast_to(b2[None, :], (8, dh))

    def kern(y1_ref, sc_ref, sh_ref, w_ref, b_ref, y_ref, st_ref):
        z = jnp.maximum(y1_ref[...] * sc_ref[...][0:1, :]
                        + sh_ref[...][0:1, :], 0.0)
        y = jnp.dot(z, w_ref[...], preferred_element_type=jnp.float32)
        y = y + b_ref[...][0:1, :]
        y_ref[...] = y
        st = _stats_block(y, dh)

        @pl.when(pl.program_id(0) == 0)
        def _():
            st_ref[...] = st

        @pl.when(pl.program_id(0) > 0)
        def _():
            st_ref[...] += st

    return pl.pallas_call(
        kern,
        grid=(E // Eb,),
        in_specs=[
            pl.BlockSpec((Eb, dh_in), lambda i: (i, 0)),
            pl.BlockSpec((8, dh_in), lambda i: (0, 0)),
            pl.BlockSpec((8, dh_in), lambda i: (0, 0)),
            pl.BlockSpec((dh_in, dh), lambda i: (0, 0)),
            pl.BlockSpec((8, dh), lambda i: (0, 0)),
        ],
        out_specs=[
            pl.BlockSpec((Eb, dh), lambda i: (i, 0)),
            pl.BlockSpec((8, dh), lambda i: (0, 0)),
        ],
        out_shape=[
            jax.ShapeDtypeStruct((E, dh), jnp.float32),
            jax.ShapeDtypeStruct((8, dh), jnp.float32),
        ],
    )(Y1, scb, shb, W2, bb)


def _tc_affine_relu_split(Y, sc, sh, Eb, epad):
    E, dh = Y.shape
    k_sl = dh // 16
    scb = jnp.broadcast_to(sc[None, :], (8, dh))
    shb = jnp.broadcast_to(sh[None, :], (8, dh))

    def kern(y_ref, sc_ref, sh_ref, *o_refs):
        m = jnp.maximum(
            y_ref[...] * sc_ref[...][0:1, :] + sh_ref[...][0:1, :], 0.0)
        for p in range(k_sl):
            o_refs[p][...] = m[:, p * 16:(p + 1) * 16]

    return pl.pallas_call(
        kern,
        grid=(E // Eb,),
        in_specs=[
            pl.BlockSpec((Eb, dh), lambda i: (i, 0)),
            pl.BlockSpec((8, dh), lambda i: (0, 0)),
            pl.BlockSpec((8, dh), lambda i: (0, 0)),
        ],
        out_specs=[pl.BlockSpec((Eb, 16), lambda i: (i, 0))
                   for _ in range(k_sl)],
        out_shape=[jax.ShapeDtypeStruct((epad, 16), jnp.float32)
                   for _ in range(k_sl)],
    )(Y, scb, shb)


def _tc_affine_relu(Y, sc, sh, Eb):
    E, dh = Y.shape
    scb = jnp.broadcast_to(sc[None, :], (8, dh))
    shb = jnp.broadcast_to(sh[None, :], (8, dh))

    def kern(y_ref, sc_ref, sh_ref, o_ref):
        o_ref[...] = jnp.maximum(
            y_ref[...] * sc_ref[...][0:1, :] + sh_ref[...][0:1, :], 0.0)

    return pl.pallas_call(
        kern,
        grid=(E // Eb,),
        in_specs=[
            pl.BlockSpec((Eb, dh), lambda i: (i, 0)),
            pl.BlockSpec((8, dh), lambda i: (0, 0)),
            pl.BlockSpec((8, dh), lambda i: (0, 0)),
        ],
        out_specs=pl.BlockSpec((Eb, dh), lambda i: (i, 0)),
        out_shape=jax.ShapeDtypeStruct((E, dh), jnp.float32),
    )(Y, scb, shb)


def _tc_combine(parts, Nb=1792):
    k_sl = len(parts)

    def kern(*refs):
        p_refs = refs[:k_sl]
        o_ref = refs[k_sl]
        cols = [p_refs[p][...] for p in range(k_sl)]
        if k_sl < 8:
            cols.append(jnp.zeros((Nb, 128 - 16 * k_sl), jnp.float32))
        o_ref[...] = jnp.concatenate(cols, axis=1)

    return pl.pallas_call(
        kern,
        grid=(NPAD // Nb,),
        in_specs=[pl.BlockSpec((Nb, 16), lambda i: (i, 0))
                  for _ in range(k_sl)],
        out_specs=pl.BlockSpec((Nb, 128), lambda i: (i, 0)),
        out_shape=jax.ShapeDtypeStruct((NPAD, 128), jnp.float32),
    )(*parts)


def _tc_head_final(pool, ncol, W2a, W2b, b2, W3, b3):
    Bp = pool.shape[0]
    b2b = jnp.broadcast_to(b2[None, :], (8, 256))
    b3b = jnp.broadcast_to(b3[None, :], (8, 128))
    W2bb = jnp.broadcast_to(W2b[None, :], (8, 256))

    def kern(s_ref, n_ref, w2a_ref, w2b_ref, b2_ref, w3_ref, b3_ref, o_ref):
        sv = s_ref[...]
        sm = sv[:, 0:384]
        cnt = sv[:, 384:385]
        pooled = sm / jnp.maximum(cnt, 1.0)
        ncon = n_ref[...][:, 0:1]
        z = jnp.dot(pooled, w2a_ref[...], preferred_element_type=jnp.float32)
        z = z + ncon * w2b_ref[...][0:1, :]
        z = jnp.maximum(z + b2_ref[...][0:1, :], 0.0)
        o = jnp.dot(z, w3_ref[...], preferred_element_type=jnp.float32)
        o = o + b3_ref[...][0:1, :]
        o_ref[...] = jax.nn.sigmoid(o)

    return pl.pallas_call(
        kern,
        out_shape=jax.ShapeDtypeStruct((Bp, 128), jnp.float32),
    )(pool, ncol, W2a, W2bb, b2b, W3, b3b)


def _affine(stats, n, g, b):
    s = stats[0]
    q = stats[1]
    mean = s / n
    var = q / n - mean * mean
    sc = g * jax.lax.rsqrt(var + EPS)
    sh = b - sc * mean
    return sc, sh


def kernel(x, edge_index, batch, Nconstituents, params):
    src = edge_index[0]
    dst = edge_index[1]
    N = x.shape[0]
    E = src.shape[0]
    B = Nconstituents.shape[0]
    Eb = 3200
    EPAD = E + 1024
    WIN = NPAD // NW

    perm = jnp.argsort(dst)
    dst_s = jnp.asarray(dst[perm], jnp.int32)
    src_s = jnp.asarray(src[perm], jnp.int32)
    bounds = jnp.arange(33, dtype=jnp.int32) * WIN
    rowptr = jnp.pad(
        jnp.searchsorted(dst_s, bounds).astype(jnp.int32), (0, 7))
    dstp = jnp.pad(dst_s, (0, EPAD - E), constant_values=2 ** 30)

    h = jnp.pad(x, ((0, NPAD - N), (0, D - x.shape[1])))
    d_true = 3
    xs = []
    for p in params["convs"]:
        W1 = p["W1"]
        W1a = jnp.pad(W1[:d_true], ((0, D - d_true), (0, 0)))
        W1b = jnp.pad(W1[d_true:], ((0, D - d_true), (0, 0)))
        Xi, Xj = _sc_gather(h, dst_s, src_s)
        Y1, st1 = _tc_mm2_stats(Xi, Xj, W1a - W1b, W1b, p["b1"], Eb)
        sc1, sh1 = _affine(st1, E, p["bn1"]["g"], p["bn1"]["b"])
        Y2, st2 = _tc_affine_mm_stats(Y1, sc1, sh1, p["W2"], p["b2"], Eb)
        sc2, sh2 = _affine(st2, E, p["bn2"]["g"], p["bn2"]["b"])
        ms = _tc_affine_relu_split(Y2, sc2, sh2, Eb, EPAD)
        parts = [_sc_scatter(m_sl, dstp, rowptr).reshape(NPAD, 16)
                 for m_sl in ms]
        h = _tc_combine(parts)
        xs.append(h[:N, :W1.shape[1]])
        d_true = W1.shape[1]

    z = jnp.concatenate(xs, axis=1)
    Y, stH = _tc_mm2_stats(z[:, :256], z[:, 256:],
                           params["seq1"]["W"][:256], params["seq1"]["W"][256:],
                           params["seq1"]["b"], 2000)
    scH, shH = _affine(stH, N, params["seq1"]["bn"]["g"],
                       params["seq1"]["bn"]["b"])
    zr = _tc_affine_relu(Y, scH, shH, 2000)

    batch_r = jnp.asarray(batch, jnp.int32).reshape(25, 1, 2000)
    pool = _tc_pool(zr, batch_r)

    Bp = 512
    ncol = jnp.zeros((Bp, 128), jnp.float32).at[:B, 0].set(Nconstituents)
    W2 = params["seq2"]["W"]
    W3 = jnp.pad(params["lin"]["W"], ((0, 0), (0, 127)))
    b3 = jnp.pad(params["lin"]["b"], (0, 127))
    out = _tc_head_final(pool, ncol, W2[:384], W2[384],
                         params["seq2"]["b"], W3, b3)
    return out[:B, 0:1]


# speedup vs baseline: 1.1111x; 1.1111x over previous
import functools

import jax
import jax.numpy as jnp
from jax import lax
from jax.experimental import pallas as pl
from jax.experimental.pallas import tpu as pltpu
from jax.experimental.pallas import tpu_sc as plsc

EPS = 1e-5
NC = 2
NS = 16
NW = NC * NS
NPAD = 50176
D = 128


def _sc_gather(h, dst, src, chunk=200):
    E = dst.shape[0]
    per_w = E // NW
    n_chunks = per_w // chunk
    mesh = plsc.VectorSubcoreMesh(core_axis_name="c", subcore_axis_name="s")

    @functools.partial(
        pl.kernel, mesh=mesh,
        out_type=(jax.ShapeDtypeStruct((E, D), jnp.float32),
                  jax.ShapeDtypeStruct((E, D), jnp.float32)),
        scratch_types=[
            pltpu.VMEM((chunk,), jnp.int32),
            pltpu.VMEM((chunk,), jnp.int32),
            pltpu.VMEM((chunk, D), jnp.float32),
            pltpu.VMEM((chunk, D), jnp.float32),
            pltpu.SemaphoreType.DMA,
            pltpu.SemaphoreType.DMA,
        ],
    )
    def k(h_hbm, dst_hbm, src_hbm, xi_hbm, xj_hbm,
          idx_i, idx_j, rows_i, rows_j, sem_i, sem_j):
        wid = lax.axis_index("s") * NC + lax.axis_index("c")
        base_w = wid * per_w

        def body(ci, _):
            base = base_w + ci * chunk
            pltpu.sync_copy(dst_hbm.at[pl.ds(base, chunk)], idx_i)
            pltpu.sync_copy(src_hbm.at[pl.ds(base, chunk)], idx_j)
            cp_i = pltpu.async_copy(h_hbm.at[idx_i], rows_i, sem_i)
            cp_j = pltpu.async_copy(h_hbm.at[idx_j], rows_j, sem_j)
            cp_i.wait()
            cp_j.wait()
            pltpu.sync_copy(rows_i, xi_hbm.at[pl.ds(base, chunk)])
            pltpu.sync_copy(rows_j, xj_hbm.at[pl.ds(base, chunk)])
            return 0

        lax.fori_loop(0, n_chunks, body, 0)

    return k(h, dst, src)


def _sc_scatter(m_flat, dstp, rowptr, dh):
    FCW = 16
    WIN = NPAD // NW
    if dh == 128:
        n_sub, subw, chunk = 3, 528, 240
    elif dh == 64:
        n_sub, subw, chunk = 1, WIN, 232
    else:
        n_sub, subw, chunk = 1, WIN, 1000
    zeros = jnp.zeros((subw * dh + 16,), jnp.float32)
    mesh = plsc.VectorSubcoreMesh(core_axis_name="c", subcore_axis_name="s")

    @functools.partial(
        pl.kernel, mesh=mesh,
        compiler_params=pltpu.CompilerParams(needs_layout_passes=False),
        out_type=jax.ShapeDtypeStruct((NPAD * dh,), jnp.float32),
        scratch_types=[
            pltpu.VMEM((chunk,), jnp.int32),
            pltpu.VMEM((chunk,), jnp.int32),
            pltpu.VMEM((chunk * dh,), jnp.float32),
            pltpu.VMEM((chunk * dh,), jnp.float32),
            pltpu.VMEM((subw * dh + 16,), jnp.float32),
            pltpu.VMEM((104,), jnp.int32),
            pltpu.SemaphoreType.DMA,
            pltpu.SemaphoreType.DMA,
            pltpu.SemaphoreType.DMA,
            pltpu.SemaphoreType.DMA,
        ],
    )
    def k(m_hbm, dst_hbm, rp_hbm, z_hbm, out_hbm,
          idx0, idx1, vals0, vals1, acc_v, rp_v, si0, si1, sv0, sv1):
        c = lax.axis_index("c")
        s = lax.axis_index("s")
        w = s * NC + c
        pltpu.sync_copy(rp_hbm, rp_v)
        iota = lax.broadcasted_iota(jnp.int32, (16,), 0)
        idx_b = (idx0, idx1)
        vals_b = (vals0, vals1)
        si_b = (si0, si1)
        sv_b = (sv0, sv1)

        def rp_read(i):
            grp = (i // 16) * 16
            vec = rp_v[pl.ds(grp, 16)]
            return jnp.sum(jnp.where(iota == i - grp, vec, 0))

        for ksub in range(n_sub):
            r0 = w * WIN + ksub * subw
            sublen = min(subw, WIN - ksub * subw)
            j = w * n_sub + ksub
            lo = rp_read(j)
            hi = rp_read(j + 1)
            lo_r = (lo // 8) * 8
            n_ch = (hi - lo_r + chunk - 1) // chunk
            pltpu.sync_copy(z_hbm, acc_v)

            def start(ci, b):
                base = lo_r + ci * chunk
                pltpu.async_copy(dst_hbm.at[pl.ds(base, chunk)],
                                 idx_b[b], si_b[b])
                pltpu.async_copy(m_hbm.at[pl.ds(base * dh, chunk * dh)],
                                 vals_b[b], sv_b[b])

            def wait(b):
                pltpu.make_async_copy(dst_hbm.at[pl.ds(0, chunk)],
                                      idx_b[b], si_b[b]).wait()
                pltpu.make_async_copy(m_hbm.at[pl.ds(0, chunk * dh)],
                                      vals_b[b], sv_b[b]).wait()

            def process(b):
                iv = idx_b[b]
                vv = vals_b[b]

                def group(g, _):
                    rows = g * 16 + iota
                    idxv = iv[pl.ds(g * 16, 16)]
                    ok = (idxv >= r0) & (idxv < r0 + sublen)
                    a = jnp.where(ok, (idxv - r0) * dh, sublen * dh)
                    radd = rows * dh
                    for jj in range(dh):
                        jf = jnp.full((16,), jj, jnp.int32)
                        v = plsc.load_gather(vv, [radd + jf])
                        plsc.addupdate_scatter(acc_v, [a + jf], v)
                    return 0

                lax.fori_loop(0, chunk // FCW, group, 0)

            start(0, 0)

            def pair(p, _):
                c0 = 2 * p
                start(c0 + 1, 1)
                wait(0)
                process(0)
                start(c0 + 2, 0)
                wait(1)
                process(1)
                return 0

            lax.fori_loop(0, (n_ch + 1) // 2, pair, 0)
            wait(0)
            pltpu.sync_copy(
                acc_v.at[pl.ds(0, sublen * dh)],
                out_hbm.at[pl.ds(r0 * dh, sublen * dh)])

    return k(m_flat, dstp, rowptr, zeros)


def _tc_pool(zr, batch_r, Nb=2000):
    N, F = zr.shape

    def kern(z_ref, b_ref, o_ref):
        brow = b_ref[...].reshape(1, Nb)
        rows = lax.broadcasted_iota(jnp.int32, (512, Nb), 0)
        oht = (rows == jnp.broadcast_to(brow, (512, Nb))).astype(jnp.float32)
        zcat = jnp.concatenate(
            [z_ref[...], jnp.ones((Nb, 128), jnp.float32)], axis=1)
        part = jnp.dot(oht, zcat, preferred_element_type=jnp.float32)

        @pl.when(pl.program_id(0) == 0)
        def _():
            o_ref[...] = part

        @pl.when(pl.program_id(0) > 0)
        def _():
            o_ref[...] += part

    return pl.pallas_call(
        kern,
        grid=(N // Nb,),
        in_specs=[
            pl.BlockSpec((Nb, F), lambda i: (i, 0)),
            pl.BlockSpec((1, 1, Nb), lambda i: (i, 0, 0)),
        ],
        out_specs=pl.BlockSpec((512, F + 128), lambda i: (0, 0)),
        out_shape=jax.ShapeDtypeStruct((512, F + 128), jnp.float32),
    )(zr, batch_r)


def _stats_block(y, dh):
    s = jnp.sum(y, axis=0, keepdims=True)
    q = jnp.sum(y * y, axis=0, keepdims=True)
    rows = lax.broadcasted_iota(jnp.int32, (8, dh), 0)
    return jnp.where(rows == 0, jnp.broadcast_to(s, (8, dh)),
                     jnp.where(rows == 1, jnp.broadcast_to(q, (8, dh)), 0.0))


def _tc_mm2_stats(Xi, Xj, Wa, Wb, b, Eb):
    E, d_i = Xi.shape
    d_j = Xj.shape[1]
    dh = Wa.shape[1]
    bb = jnp.broadcast_to(b[None, :], (8, dh))

    def kern(xi_ref, xj_ref, wa_ref, wb_ref, b_ref, y_ref, st_ref):
        y = jnp.dot(xi_ref[...], wa_ref[...],
                    preferred_element_type=jnp.float32)
        y = y + jnp.dot(xj_ref[...], wb_ref[...],
                        preferred_element_type=jnp.float32)
        y = y + b_ref[...][0:1, :]
        y_ref[...] = y
        st = _stats_block(y, dh)

        @pl.when(pl.program_id(0) == 0)
        def _():
            st_ref[...] = st

        @pl.when(pl.program_id(0) > 0)
        def _():
            st_ref[...] += st

    return pl.pallas_call(
        kern,
        grid=(E // Eb,),
        in_specs=[
            pl.BlockSpec((Eb, d_i), lambda i: (i, 0)),
            pl.BlockSpec((Eb, d_j), lambda i: (i, 0)),
            pl.BlockSpec((d_i, dh), lambda i: (0, 0)),
            pl.BlockSpec((d_j, dh), lambda i: (0, 0)),
            pl.BlockSpec((8, dh), lambda i: (0, 0)),
        ],
        out_specs=[
            pl.BlockSpec((Eb, dh), lambda i: (i, 0)),
            pl.BlockSpec((8, dh), lambda i: (0, 0)),
        ],
        out_shape=[
            jax.ShapeDtypeStruct((E, dh), jnp.float32),
            jax.ShapeDtypeStruct((8, dh), jnp.float32),
        ],
    )(Xi, Xj, Wa, Wb, bb)


def _tc_affine_mm_stats(Y1, sc, sh, W2, b2, Eb):
    E, dh_in = Y1.shape
    dh = W2.shape[1]
    scb = jnp.broadcast_to(sc[None, :], (8, dh_in))
    shb = jnp.broadcast_to(sh[None, :], (8, dh_in))
    bb = jnp.broadcast_to(b2[None, :], (8, dh))

    def kern(y1_ref, sc_ref, sh_ref, w_ref, b_ref, y_ref, st_ref):
        z = jnp.maximum(y1_ref[...] * sc_ref[...][0:1, :]
                        + sh_ref[...][0:1, :], 0.0)
        y = jnp.dot(z, w_ref[...], preferred_element_type=jnp.float32)
        y = y + b_ref[...][0:1, :]
        y_ref[...] = y
        st = _stats_block(y, dh)

        @pl.when(pl.program_id(0) == 0)
        def _():
            st_ref[...] = st

        @pl.when(pl.program_id(0) > 0)
        def _():
            st_ref[...] += st

    return pl.pallas_call(
        kern,
        grid=(E // Eb,),
        in_specs=[
            pl.BlockSpec((Eb, dh_in), lambda i: (i, 0)),
            pl.BlockSpec((8, dh_in), lambda i: (0, 0)),
            pl.BlockSpec((8, dh_in), lambda i: (0, 0)),
            pl.BlockSpec((dh_in, dh), lambda i: (0, 0)),
            pl.BlockSpec((8, dh), lambda i: (0, 0)),
        ],
        out_specs=[
            pl.BlockSpec((Eb, dh), lambda i: (i, 0)),
            pl.BlockSpec((8, dh), lambda i: (0, 0)),
        ],
        out_shape=[
            jax.ShapeDtypeStruct((E, dh), jnp.float32),
            jax.ShapeDtypeStruct((8, dh), jnp.float32),
        ],
    )(Y1, scb, shb, W2, bb)


def _tc_affine_relu_flat(Y, sc, sh, Eb, epad):
    E, dh = Y.shape
    scb = jnp.broadcast_to(sc[None, :], (8, dh))
    shb = jnp.broadcast_to(sh[None, :], (8, dh))

    def kern(y_ref, sc_ref, sh_ref, o_ref):
        o_ref[...] = jnp.maximum(
            y_ref[...] * sc_ref[...][0:1, :] + sh_ref[...][0:1, :], 0.0)

    return pl.pallas_call(
        kern,
        grid=(E // Eb,),
        in_specs=[
            pl.BlockSpec((Eb, dh), lambda i: (i, 0)),
            pl.BlockSpec((8, dh), lambda i: (0, 0)),
            pl.BlockSpec((8, dh), lambda i: (0, 0)),
        ],
        out_specs=pl.BlockSpec((Eb, dh), lambda i: (i, 0)),
        out_shape=jax.ShapeDtypeStruct((epad, dh), jnp.float32),
    )(Y, scb, shb)


def _tc_affine_relu(Y, sc, sh, Eb):
    E, dh = Y.shape
    scb = jnp.broadcast_to(sc[None, :], (8, dh))
    shb = jnp.broadcast_to(sh[None, :], (8, dh))

    def kern(y_ref, sc_ref, sh_ref, o_ref):
        o_ref[...] = jnp.maximum(
            y_ref[...] * sc_ref[...][0:1, :] + sh_ref[...][0:1, :], 0.0)

    return pl.pallas_call(
        kern,
        grid=(E // Eb,),
        in_specs=[
            pl.BlockSpec((Eb, dh), lambda i: (i, 0)),
            pl.BlockSpec((8, dh), lambda i: (0, 0)),
            pl.BlockSpec((8, dh), lambda i: (0, 0)),
        ],
        out_specs=pl.BlockSpec((Eb, dh), lambda i: (i, 0)),
        out_shape=jax.ShapeDtypeStruct((E, dh), jnp.float32),
    )(Y, scb, shb)


def _tc_combine(part, Nb=1792):
    dh = part.shape[1]

    def kern(p_ref, o_ref):
        v = p_ref[...]
        if dh < 128:
            v = jnp.concatenate(
                [v, jnp.zeros((Nb, 128 - dh), jnp.float32)], axis=1)
        o_ref[...] = v

    return pl.pallas_call(
        kern,
        grid=(NPAD // Nb,),
        in_specs=[pl.BlockSpec((Nb, dh), lambda i: (i, 0))],
        out_specs=pl.BlockSpec((Nb, 128), lambda i: (i, 0)),
        out_shape=jax.ShapeDtypeStruct((NPAD, 128), jnp.float32),
    )(part)


def _tc_head_final(pool, ncol, W2a, W2b, b2, W3, b3):
    Bp = pool.shape[0]
    b2b = jnp.broadcast_to(b2[None, :], (8, 256))
    b3b = jnp.broadcast_to(b3[None, :], (8, 128))
    W2bb = jnp.broadcast_to(W2b[None, :], (8, 256))

    def kern(s_ref, n_ref, w2a_ref, w2b_ref, b2_ref, w3_ref, b3_ref, o_ref):
        sv = s_ref[...]
        sm = sv[:, 0:384]
        cnt = sv[:, 384:385]
        pooled = sm / jnp.maximum(cnt, 1.0)
        ncon = n_ref[...][:, 0:1]
        z = jnp.dot(pooled, w2a_ref[...], preferred_element_type=jnp.float32)
        z = z + ncon * w2b_ref[...][0:1, :]
        z = jnp.maximum(z + b2_ref[...][0:1, :], 0.0)
        o = jnp.dot(z, w3_ref[...], preferred_element_type=jnp.float32)
        o = o + b3_ref[...][0:1, :]
        o_ref[...] = jax.nn.sigmoid(o)

    return pl.pallas_call(
        kern,
        out_shape=jax.ShapeDtypeStruct((Bp, 128), jnp.float32),
    )(pool, ncol, W2a, W2bb, b2b, W3, b3b)


def _affine(stats, n, g, b):
    s = stats[0]
    q = stats[1]
    mean = s / n
    var = q / n - mean * mean
    sc = g * jax.lax.rsqrt(var + EPS)
    sh = b - sc * mean
    return sc, sh


def kernel(x, edge_index, batch, Nconstituents, params):
    src = edge_index[0]
    dst = edge_index[1]
    N = x.shape[0]
    E = src.shape[0]
    B = Nconstituents.shape[0]
    Eb = 3200
    EPAD = E + 2048
    WIN = NPAD // NW

    perm = jnp.argsort(dst)
    dst_s = jnp.asarray(dst[perm], jnp.int32)
    src_s = jnp.asarray(src[perm], jnp.int32)
    bsub = jnp.minimum(jnp.arange(3, dtype=jnp.int32) * 528, WIN)
    bounds = (jnp.arange(32, dtype=jnp.int32)[:, None] * WIN
              + bsub[None, :]).reshape(-1)
    bounds = jnp.concatenate(
        [bounds, jnp.full((1,), NPAD, jnp.int32)])
    rowptr = jnp.pad(
        jnp.searchsorted(dst_s, bounds).astype(jnp.int32), (0, 7))
    bounds1 = jnp.arange(33, dtype=jnp.int32) * WIN
    rowptr1 = jnp.pad(
        jnp.searchsorted(dst_s, bounds1).astype(jnp.int32), (0, 71))
    dstp = jnp.pad(dst_s, (0, EPAD - E), constant_values=2 ** 30)

    h = jnp.pad(x, ((0, NPAD - N), (0, D - x.shape[1])))
    d_true = 3
    xs = []
    for p in params["convs"]:
        W1 = p["W1"]
        W1a = jnp.pad(W1[:d_true], ((0, D - d_true), (0, 0)))
        W1b = jnp.pad(W1[d_true:], ((0, D - d_true), (0, 0)))
        Xi, Xj = _sc_gather(h, dst_s, src_s)
        Y1, st1 = _tc_mm2_stats(Xi, Xj, W1a - W1b, W1b, p["b1"], Eb)
        sc1, sh1 = _affine(st1, E, p["bn1"]["g"], p["bn1"]["b"])
        Y2, st2 = _tc_affine_mm_stats(Y1, sc1, sh1, p["W2"], p["b2"], Eb)
        sc2, sh2 = _affine(st2, E, p["bn2"]["g"], p["bn2"]["b"])
        dh = W1.shape[1]
        m2d = _tc_affine_relu_flat(Y2, sc2, sh2, Eb, EPAD)
        rp = rowptr if dh == 128 else rowptr1
        flat = _sc_scatter(m2d.reshape(EPAD * dh), dstp, rp, dh)
        h = _tc_combine(flat.reshape(NPAD, dh))
        xs.append(h[:N, :W1.shape[1]])
        d_true = W1.shape[1]

    z = jnp.concatenate(xs, axis=1)
    Y, stH = _tc_mm2_stats(z[:, :256], z[:, 256:],
                           params["seq1"]["W"][:256], params["seq1"]["W"][256:],
                           params["seq1"]["b"], 2000)
    scH, shH = _affine(stH, N, params["seq1"]["bn"]["g"],
                       params["seq1"]["bn"]["b"])
    zr = _tc_affine_relu(Y, scH, shH, 2000)

    batch_r = jnp.asarray(batch, jnp.int32).reshape(25, 1, 2000)
    pool = _tc_pool(zr, batch_r)

    Bp = 512
    ncol = jnp.zeros((Bp, 128), jnp.float32).at[:B, 0].set(Nconstituents)
    W2 = params["seq2"]["W"]
    W3 = jnp.pad(params["lin"]["W"], ((0, 0), (0, 127)))
    b3 = jnp.pad(params["lin"]["b"], (0, 127))
    out = _tc_head_final(pool, ncol, W2[:384], W2[384],
                         params["seq2"]["b"], W3, b3)
    return out[:B, 0:1]


# speedup vs baseline: 1.1189x; 1.0070x over previous
import functools

import jax
import jax.numpy as jnp
from jax import lax
from jax.experimental import pallas as pl
from jax.experimental.pallas import tpu as pltpu
from jax.experimental.pallas import tpu_sc as plsc

EPS = 1e-5
NC = 2
NS = 16
NW = NC * NS
NPAD = 50176
D = 128


def _sc_gather(h, dst, src, chunk=200):
    E = dst.shape[0]
    per_w = E // NW
    n_chunks = per_w // chunk
    mesh = plsc.VectorSubcoreMesh(core_axis_name="c", subcore_axis_name="s")

    @functools.partial(
        pl.kernel, mesh=mesh,
        out_type=(jax.ShapeDtypeStruct((E, D), jnp.float32),
                  jax.ShapeDtypeStruct((E, D), jnp.float32)),
        scratch_types=[
            pltpu.VMEM((chunk,), jnp.int32),
            pltpu.VMEM((chunk,), jnp.int32),
            pltpu.VMEM((chunk, D), jnp.float32),
            pltpu.VMEM((chunk, D), jnp.float32),
            pltpu.SemaphoreType.DMA,
            pltpu.SemaphoreType.DMA,
        ],
    )
    def k(h_hbm, dst_hbm, src_hbm, xi_hbm, xj_hbm,
          idx_i, idx_j, rows_i, rows_j, sem_i, sem_j):
        wid = lax.axis_index("s") * NC + lax.axis_index("c")
        base_w = wid * per_w

        def body(ci, _):
            base = base_w + ci * chunk
            pltpu.sync_copy(dst_hbm.at[pl.ds(base, chunk)], idx_i)
            pltpu.sync_copy(src_hbm.at[pl.ds(base, chunk)], idx_j)
            cp_i = pltpu.async_copy(h_hbm.at[idx_i], rows_i, sem_i)
            cp_j = pltpu.async_copy(h_hbm.at[idx_j], rows_j, sem_j)
            cp_i.wait()
            cp_j.wait()
            pltpu.sync_copy(rows_i, xi_hbm.at[pl.ds(base, chunk)])
            pltpu.sync_copy(rows_j, xj_hbm.at[pl.ds(base, chunk)])
            return 0

        lax.fori_loop(0, n_chunks, body, 0)

    return k(h, dst, src)


def _sc_scatter(m_flat, dstp, rowptr, dh):
    FCW = 16
    WIN = NPAD // NW
    if dh == 128:
        n_sub, subw, chunk = 3, 528, 224
    elif dh == 64:
        n_sub, subw, chunk = 1, WIN, 224
    else:
        n_sub, subw, chunk = 1, WIN, 992
    zeros = jnp.zeros((subw * dh + 16,), jnp.float32)
    mesh = plsc.VectorSubcoreMesh(core_axis_name="c", subcore_axis_name="s")

    @functools.partial(
        pl.kernel, mesh=mesh,
        compiler_params=pltpu.CompilerParams(needs_layout_passes=False),
        out_type=jax.ShapeDtypeStruct((NPAD * dh,), jnp.float32),
        scratch_types=[
            pltpu.VMEM((chunk,), jnp.int32),
            pltpu.VMEM((chunk,), jnp.int32),
            pltpu.VMEM((chunk * dh,), jnp.float32),
            pltpu.VMEM((chunk * dh,), jnp.float32),
            pltpu.VMEM((subw * dh + 16,), jnp.float32),
            pltpu.VMEM((104,), jnp.int32),
            pltpu.SemaphoreType.DMA,
            pltpu.SemaphoreType.DMA,
            pltpu.SemaphoreType.DMA,
            pltpu.SemaphoreType.DMA,
        ],
    )
    def k(m_hbm, dst_hbm, rp_hbm, z_hbm, out_hbm,
          idx0, idx1, vals0, vals1, acc_v, rp_v, si0, si1, sv0, sv1):
        c = lax.axis_index("c")
        s = lax.axis_index("s")
        w = s * NC + c
        pltpu.sync_copy(rp_hbm, rp_v)
        iota = lax.broadcasted_iota(jnp.int32, (16,), 0)
        idx_b = (idx0, idx1)
        vals_b = (vals0, vals1)
        si_b = (si0, si1)
        sv_b = (sv0, sv1)

        def rp_read(i):
            grp = (i // 16) * 16
            vec = rp_v[pl.ds(grp, 16)]
            return jnp.sum(jnp.where(iota == i - grp, vec, 0))

        for ksub in range(n_sub):
            r0 = w * WIN + ksub * subw
            sublen = min(subw, WIN - ksub * subw)
            j = w * n_sub + ksub
            lo = rp_read(j)
            hi = rp_read(j + 1)
            lo_r = (lo // 8) * 8
            n_ch = (hi - lo_r + chunk - 1) // chunk
            pltpu.sync_copy(z_hbm, acc_v)

            def start(ci, b):
                base = lo_r + ci * chunk
                pltpu.async_copy(dst_hbm.at[pl.ds(base, chunk)],
                                 idx_b[b], si_b[b])
                pltpu.async_copy(m_hbm.at[pl.ds(base * dh, chunk * dh)],
                                 vals_b[b], sv_b[b])

            def wait(b):
                pltpu.make_async_copy(dst_hbm.at[pl.ds(0, chunk)],
                                      idx_b[b], si_b[b]).wait()
                pltpu.make_async_copy(m_hbm.at[pl.ds(0, chunk * dh)],
                                      vals_b[b], sv_b[b]).wait()

            def process(b):
                iv = idx_b[b]
                vv = vals_b[b]

                def group(g, _):
                    rows = g * 32 + iota
                    idxv = iv[pl.ds(g * 32, 16)]
                    ok = (idxv >= r0) & (idxv < r0 + sublen)
                    a = jnp.where(ok, (idxv - r0) * dh, sublen * dh)
                    radd = rows * dh
                    rows2 = rows + 16
                    idxv2 = iv[pl.ds(g * 32 + 16, 16)]
                    ok2 = (idxv2 >= r0) & (idxv2 < r0 + sublen)
                    a2 = jnp.where(ok2, (idxv2 - r0) * dh, sublen * dh)
                    radd2 = rows2 * dh
                    for jj in range(dh):
                        jf = jnp.full((16,), jj, jnp.int32)
                        v = plsc.load_gather(vv, [radd + jf])
                        v2 = plsc.load_gather(vv, [radd2 + jf])
                        plsc.addupdate_scatter(acc_v, [a + jf], v)
                        plsc.addupdate_scatter(acc_v, [a2 + jf], v2)
                    return 0

                lax.fori_loop(0, chunk // 32, group, 0)

            start(0, 0)

            def pair(p, _):
                c0 = 2 * p
                start(c0 + 1, 1)
                wait(0)
                process(0)
                start(c0 + 2, 0)
                wait(1)
                process(1)
                return 0

            lax.fori_loop(0, (n_ch + 1) // 2, pair, 0)
            wait(0)
            pltpu.sync_copy(
                acc_v.at[pl.ds(0, sublen * dh)],
                out_hbm.at[pl.ds(r0 * dh, sublen * dh)])

    return k(m_flat, dstp, rowptr, zeros)


def _tc_pool(zr, batch_r, Nb=2000):
    N, F = zr.shape

    def kern(z_ref, b_ref, o_ref):
        brow = b_ref[...].reshape(1, Nb)
        rows = lax.broadcasted_iota(jnp.int32, (512, Nb), 0)
        oht = (rows == jnp.broadcast_to(brow, (512, Nb))).astype(jnp.float32)
        zcat = jnp.concatenate(
            [z_ref[...], jnp.ones((Nb, 128), jnp.float32)], axis=1)
        part = jnp.dot(oht, zcat, preferred_element_type=jnp.float32)

        @pl.when(pl.program_id(0) == 0)
        def _():
            o_ref[...] = part

        @pl.when(pl.program_id(0) > 0)
        def _():
            o_ref[...] += part

    return pl.pallas_call(
        kern,
        grid=(N // Nb,),
        in_specs=[
            pl.BlockSpec((Nb, F), lambda i: (i, 0)),
            pl.BlockSpec((1, 1, Nb), lambda i: (i, 0, 0)),
        ],
        out_specs=pl.BlockSpec((512, F + 128), lambda i: (0, 0)),
        out_shape=jax.ShapeDtypeStruct((512, F + 128), jnp.float32),
    )(zr, batch_r)


def _stats_block(y, dh):
    s = jnp.sum(y, axis=0, keepdims=True)
    q = jnp.sum(y * y, axis=0, keepdims=True)
    rows = lax.broadcasted_iota(jnp.int32, (8, dh), 0)
    return jnp.where(rows == 0, jnp.broadcast_to(s, (8, dh)),
                     jnp.where(rows == 1, jnp.broadcast_to(q, (8, dh)), 0.0))


def _tc_mm2_stats(Xi, Xj, Wa, Wb, b, Eb):
    E, d_i = Xi.shape
    d_j = Xj.shape[1]
    dh = Wa.shape[1]
    bb = jnp.broadcast_to(b[None, :], (8, dh))

    def kern(xi_ref, xj_ref, wa_ref, wb_ref, b_ref, y_ref, st_ref):
        y = jnp.dot(xi_ref[...], wa_ref[...],
                    preferred_element_type=jnp.float32)
        y = y + jnp.dot(xj_ref[...], wb_ref[...],
                        preferred_element_type=jnp.float32)
        y = y + b_ref[...][0:1, :]
        y_ref[...] = y
        st = _stats_block(y, dh)

        @pl.when(pl.program_id(0) == 0)
        def _():
            st_ref[...] = st

        @pl.when(pl.program_id(0) > 0)
        def _():
            st_ref[...] += st

    return pl.pallas_call(
        kern,
        grid=(E // Eb,),
        in_specs=[
            pl.BlockSpec((Eb, d_i), lambda i: (i, 0)),
            pl.BlockSpec((Eb, d_j), lambda i: (i, 0)),
            pl.BlockSpec((d_i, dh), lambda i: (0, 0)),
            pl.BlockSpec((d_j, dh), lambda i: (0, 0)),
            pl.BlockSpec((8, dh), lambda i: (0, 0)),
        ],
        out_specs=[
            pl.BlockSpec((Eb, dh), lambda i: (i, 0)),
            pl.BlockSpec((8, dh), lambda i: (0, 0)),
        ],
        out_shape=[
            jax.ShapeDtypeStruct((E, dh), jnp.float32),
            jax.ShapeDtypeStruct((8, dh), jnp.float32),
        ],
    )(Xi, Xj, Wa, Wb, bb)


def _tc_affine_mm_stats(Y1, sc, sh, W2, b2, Eb):
    E, dh_in = Y1.shape
    dh = W2.shape[1]
    scb = jnp.broadcast_to(sc[None, :], (8, dh_in))
    shb = jnp.broadcast_to(sh[None, :], (8, dh_in))
    bb = jnp.broadcast_to(b2[None, :], (8, dh))

    def kern(y1_ref, sc_ref, sh_ref, w_ref, b_ref, y_ref, st_ref):
        z = jnp.maximum(y1_ref[...] * sc_ref[...][0:1, :]
                        + sh_ref[...][0:1, :], 0.0)
        y = jnp.dot(z, w_ref[...], preferred_element_type=jnp.float32)
        y = y + b_ref[...][0:1, :]
        y_ref[...] = y
        st = _stats_block(y, dh)

        @pl.when(pl.program_id(0) == 0)
        def _():
            st_ref[...] = st

        @pl.when(pl.program_id(0) > 0)
        def _():
            st_ref[...] += st

    return pl.pallas_call(
        kern,
        grid=(E // Eb,),
        in_specs=[
            pl.BlockSpec((Eb, dh_in), lambda i: (i, 0)),
            pl.BlockSpec((8, dh_in), lambda i: (0, 0)),
            pl.BlockSpec((8, dh_in), lambda i: (0, 0)),
            pl.BlockSpec((dh_in, dh), lambda i: (0, 0)),
            pl.BlockSpec((8, dh), lambda i: (0, 0)),
        ],
        out_specs=[
            pl.BlockSpec((Eb, dh), lambda i: (i, 0)),
            pl.BlockSpec((8, dh), lambda i: (0, 0)),
        ],
        out_shape=[
            jax.ShapeDtypeStruct((E, dh), jnp.float32),
            jax.ShapeDtypeStruct((8, dh), jnp.float32),
        ],
    )(Y1, scb, shb, W2, bb)


def _tc_affine_relu_flat(Y, sc, sh, Eb, epad):
    E, dh = Y.shape
    scb = jnp.broadcast_to(sc[None, :], (8, dh))
    shb = jnp.broadcast_to(sh[None, :], (8, dh))

    def kern(y_ref, sc_ref, sh_ref, o_ref):
        o_ref[...] = jnp.maximum(
            y_ref[...] * sc_ref[...][0:1, :] + sh_ref[...][0:1, :], 0.0)

    return pl.pallas_call(
        kern,
        grid=(E // Eb,),
        in_specs=[
            pl.BlockSpec((Eb, dh), lambda i: (i, 0)),
            pl.BlockSpec((8, dh), lambda i: (0, 0)),
            pl.BlockSpec((8, dh), lambda i: (0, 0)),
        ],
        out_specs=pl.BlockSpec((Eb, dh), lambda i: (i, 0)),
        out_shape=jax.ShapeDtypeStruct((epad, dh), jnp.float32),
    )(Y, scb, shb)


def _tc_affine_relu(Y, sc, sh, Eb):
    E, dh = Y.shape
    scb = jnp.broadcast_to(sc[None, :], (8, dh))
    shb = jnp.broadcast_to(sh[None, :], (8, dh))

    def kern(y_ref, sc_ref, sh_ref, o_ref):
        o_ref[...] = jnp.maximum(
            y_ref[...] * sc_ref[...][0:1, :] + sh_ref[...][0:1, :], 0.0)

    return pl.pallas_call(
        kern,
        grid=(E // Eb,),
        in_specs=[
            pl.BlockSpec((Eb, dh), lambda i: (i, 0)),
            pl.BlockSpec((8, dh), lambda i: (0, 0)),
            pl.BlockSpec((8, dh), lambda i: (0, 0)),
        ],
        out_specs=pl.BlockSpec((Eb, dh), lambda i: (i, 0)),
        out_shape=jax.ShapeDtypeStruct((E, dh), jnp.float32),
    )(Y, scb, shb)


def _tc_combine(part, Nb=1792):
    dh = part.shape[1]

    def kern(p_ref, o_ref):
        v = p_ref[...]
        if dh < 128:
            v = jnp.concatenate(
                [v, jnp.zeros((Nb, 128 - dh), jnp.float32)], axis=1)
        o_ref[...] = v

    return pl.pallas_call(
        kern,
        grid=(NPAD // Nb,),
        in_specs=[pl.BlockSpec((Nb, dh), lambda i: (i, 0))],
        out_specs=pl.BlockSpec((Nb, 128), lambda i: (i, 0)),
        out_shape=jax.ShapeDtypeStruct((NPAD, 128), jnp.float32),
    )(part)


def _tc_head_final(pool, ncol, W2a, W2b, b2, W3, b3):
    Bp = pool.shape[0]
    b2b = jnp.broadcast_to(b2[None, :], (8, 256))
    b3b = jnp.broadcast_to(b3[None, :], (8, 128))
    W2bb = jnp.broadcast_to(W2b[None, :], (8, 256))

    def kern(s_ref, n_ref, w2a_ref, w2b_ref, b2_ref, w3_ref, b3_ref, o_ref):
        sv = s_ref[...]
        sm = sv[:, 0:384]
        cnt = sv[:, 384:385]
        pooled = sm / jnp.maximum(cnt, 1.0)
        ncon = n_ref[...][:, 0:1]
        z = jnp.dot(pooled, w2a_ref[...], preferred_element_type=jnp.float32)
        z = z + ncon * w2b_ref[...][0:1, :]
        z = jnp.maximum(z + b2_ref[...][0:1, :], 0.0)
        o = jnp.dot(z, w3_ref[...], preferred_element_type=jnp.float32)
        o = o + b3_ref[...][0:1, :]
        o_ref[...] = jax.nn.sigmoid(o)

    return pl.pallas_call(
        kern,
        out_shape=jax.ShapeDtypeStruct((Bp, 128), jnp.float32),
    )(pool, ncol, W2a, W2bb, b2b, W3, b3b)


def _affine(stats, n, g, b):
    s = stats[0]
    q = stats[1]
    mean = s / n
    var = q / n - mean * mean
    sc = g * jax.lax.rsqrt(var + EPS)
    sh = b - sc * mean
    return sc, sh


def kernel(x, edge_index, batch, Nconstituents, params):
    src = edge_index[0]
    dst = edge_index[1]
    N = x.shape[0]
    E = src.shape[0]
    B = Nconstituents.shape[0]
    Eb = 3200
    EPAD = E + 2048
    WIN = NPAD // NW

    perm = jnp.argsort(dst)
    dst_s = jnp.asarray(dst[perm], jnp.int32)
    src_s = jnp.asarray(src[perm], jnp.int32)
    bsub = jnp.minimum(jnp.arange(3, dtype=jnp.int32) * 528, WIN)
    bounds = (jnp.arange(32, dtype=jnp.int32)[:, None] * WIN
              + bsub[None, :]).reshape(-1)
    bounds = jnp.concatenate(
        [bounds, jnp.full((1,), NPAD, jnp.int32)])
    rowptr = jnp.pad(
        jnp.searchsorted(dst_s, bounds).astype(jnp.int32), (0, 7))
    bounds1 = jnp.arange(33, dtype=jnp.int32) * WIN
    rowptr1 = jnp.pad(
        jnp.searchsorted(dst_s, bounds1).astype(jnp.int32), (0, 71))
    dstp = jnp.pad(dst_s, (0, EPAD - E), constant_values=2 ** 30)

    h = jnp.pad(x, ((0, NPAD - N), (0, D - x.shape[1])))
    d_true = 3
    xs = []
    for p in params["convs"]:
        W1 = p["W1"]
        W1a = jnp.pad(W1[:d_true], ((0, D - d_true), (0, 0)))
        W1b = jnp.pad(W1[d_true:], ((0, D - d_true), (0, 0)))
        Xi, Xj = _sc_gather(h, dst_s, src_s)
        Y1, st1 = _tc_mm2_stats(Xi, Xj, W1a - W1b, W1b, p["b1"], Eb)
        sc1, sh1 = _affine(st1, E, p["bn1"]["g"], p["bn1"]["b"])
        Y2, st2 = _tc_affine_mm_stats(Y1, sc1, sh1, p["W2"], p["b2"], Eb)
        sc2, sh2 = _affine(st2, E, p["bn2"]["g"], p["bn2"]["b"])
        dh = W1.shape[1]
        m2d = _tc_affine_relu_flat(Y2, sc2, sh2, Eb, EPAD)
        rp = rowptr if dh == 128 else rowptr1
        flat = _sc_scatter(m2d.reshape(EPAD * dh), dstp, rp, dh)
        h = _tc_combine(flat.reshape(NPAD, dh))
        xs.append(h[:N, :W1.shape[1]])
        d_true = W1.shape[1]

    z = jnp.concatenate(xs, axis=1)
    Y, stH = _tc_mm2_stats(z[:, :256], z[:, 256:],
                           params["seq1"]["W"][:256], params["seq1"]["W"][256:],
                           params["seq1"]["b"], 2000)
    scH, shH = _affine(stH, N, params["seq1"]["bn"]["g"],
                       params["seq1"]["bn"]["b"])
    zr = _tc_affine_relu(Y, scH, shH, 2000)

    batch_r = jnp.asarray(batch, jnp.int32).reshape(25, 1, 2000)
    pool = _tc_pool(zr, batch_r)

    Bp = 512
    ncol = jnp.zeros((Bp, 128), jnp.float32).at[:B, 0].set(Nconstituents)
    W2 = params["seq2"]["W"]
    W3 = jnp.pad(params["lin"]["W"], ((0, 0), (0, 127)))
    b3 = jnp.pad(params["lin"]["b"], (0, 127))
    out = _tc_head_final(pool, ncol, W2[:384], W2[384],
                         params["seq2"]["b"], W3, b3)
    return out[:B, 0:1]


# speedup vs baseline: 1.2274x; 1.0969x over previous
import functools

import jax
import jax.numpy as jnp
from jax import lax
from jax.experimental import pallas as pl
from jax.experimental.pallas import tpu as pltpu
from jax.experimental.pallas import tpu_sc as plsc

EPS = 1e-5
NC = 2
NS = 16
NW = NC * NS
NPAD = 50176
D = 128


def _sc_gather(h, dst, src, chunk=200):
    E = dst.shape[0]
    per_w = E // NW
    n_chunks = per_w // chunk
    mesh = plsc.VectorSubcoreMesh(core_axis_name="c", subcore_axis_name="s")

    @functools.partial(
        pl.kernel, mesh=mesh,
        out_type=(jax.ShapeDtypeStruct((E, D), jnp.float32),
                  jax.ShapeDtypeStruct((E, D), jnp.float32)),
        scratch_types=[
            pltpu.VMEM((chunk,), jnp.int32),
            pltpu.VMEM((chunk,), jnp.int32),
            pltpu.VMEM((chunk, D), jnp.float32),
            pltpu.VMEM((chunk, D), jnp.float32),
            pltpu.SemaphoreType.DMA,
            pltpu.SemaphoreType.DMA,
        ],
    )
    def k(h_hbm, dst_hbm, src_hbm, xi_hbm, xj_hbm,
          idx_i, idx_j, rows_i, rows_j, sem_i, sem_j):
        wid = lax.axis_index("s") * NC + lax.axis_index("c")
        base_w = wid * per_w

        def body(ci, _):
            base = base_w + ci * chunk
            pltpu.sync_copy(dst_hbm.at[pl.ds(base, chunk)], idx_i)
            pltpu.sync_copy(src_hbm.at[pl.ds(base, chunk)], idx_j)
            cp_i = pltpu.async_copy(h_hbm.at[idx_i], rows_i, sem_i)
            cp_j = pltpu.async_copy(h_hbm.at[idx_j], rows_j, sem_j)
            cp_i.wait()
            cp_j.wait()
            pltpu.sync_copy(rows_i, xi_hbm.at[pl.ds(base, chunk)])
            pltpu.sync_copy(rows_j, xj_hbm.at[pl.ds(base, chunk)])
            return 0

        lax.fori_loop(0, n_chunks, body, 0)

    return k(h, dst, src)


def _sc_scatter(m_flat, dstp, rowptr, dh):
    FCW = 16
    WIN = NPAD // NW
    if dh == 128:
        n_sub, subw, chunk = 3, 528, 224
    elif dh == 64:
        n_sub, subw, chunk = 1, WIN, 224
    else:
        n_sub, subw, chunk = 1, WIN, 992
    zeros = jnp.zeros((subw * dh + 16,), jnp.float32)
    mesh = plsc.VectorSubcoreMesh(core_axis_name="c", subcore_axis_name="s")

    @functools.partial(
        pl.kernel, mesh=mesh,
        compiler_params=pltpu.CompilerParams(needs_layout_passes=False),
        out_type=jax.ShapeDtypeStruct((NPAD * dh,), jnp.float32),
        scratch_types=[
            pltpu.VMEM((chunk,), jnp.int32),
            pltpu.VMEM((chunk,), jnp.int32),
            pltpu.VMEM((chunk * dh,), jnp.float32),
            pltpu.VMEM((chunk * dh,), jnp.float32),
            pltpu.VMEM((subw * dh + 16,), jnp.float32),
            pltpu.VMEM((104,), jnp.int32),
            pltpu.SemaphoreType.DMA,
            pltpu.SemaphoreType.DMA,
            pltpu.SemaphoreType.DMA,
            pltpu.SemaphoreType.DMA,
        ],
    )
    def k(m_hbm, dst_hbm, rp_hbm, z_hbm, out_hbm,
          idx0, idx1, vals0, vals1, acc_v, rp_v, si0, si1, sv0, sv1):
        c = lax.axis_index("c")
        s = lax.axis_index("s")
        w = s * NC + c
        pltpu.sync_copy(rp_hbm, rp_v)
        iota = lax.broadcasted_iota(jnp.int32, (16,), 0)
        idx_b = (idx0, idx1)
        vals_b = (vals0, vals1)
        si_b = (si0, si1)
        sv_b = (sv0, sv1)

        def rp_read(i):
            grp = (i // 16) * 16
            vec = rp_v[pl.ds(grp, 16)]
            return jnp.sum(jnp.where(iota == i - grp, vec, 0))

        for ksub in range(n_sub):
            r0 = w * WIN + ksub * subw
            sublen = min(subw, WIN - ksub * subw)
            j = w * n_sub + ksub
            lo = rp_read(j)
            hi = rp_read(j + 1)
            lo_r = (lo // 8) * 8
            n_ch = (hi - lo_r + chunk - 1) // chunk
            pltpu.sync_copy(z_hbm, acc_v)

            def start(ci, b):
                base = lo_r + ci * chunk
                pltpu.async_copy(dst_hbm.at[pl.ds(base, chunk)],
                                 idx_b[b], si_b[b])
                pltpu.async_copy(m_hbm.at[pl.ds(base * dh, chunk * dh)],
                                 vals_b[b], sv_b[b])

            def wait(b):
                pltpu.make_async_copy(dst_hbm.at[pl.ds(0, chunk)],
                                      idx_b[b], si_b[b]).wait()
                pltpu.make_async_copy(m_hbm.at[pl.ds(0, chunk * dh)],
                                      vals_b[b], sv_b[b]).wait()

            def process(b):
                iv = idx_b[b]
                vv = vals_b[b]
                ng = chunk // 16

                def group(g, _):
                    pos = g + iota * ng
                    idxv = plsc.load_gather(iv, [pos])
                    ok = (idxv >= r0) & (idxv < r0 + sublen)
                    a = jnp.where(ok, (idxv - r0) * dh, sublen * dh)
                    radd = pos * dh
                    for jj in range(dh):
                        jf = jnp.full((16,), jj, jnp.int32)
                        v = plsc.load_gather(vv, [radd + jf])
                        plsc.addupdate_scatter(acc_v, [a + jf], v)
                    return 0

                lax.fori_loop(0, ng, group, 0)

            start(0, 0)

            def pair(p, _):
                c0 = 2 * p
                start(c0 + 1, 1)
                wait(0)
                process(0)
                start(c0 + 2, 0)
                wait(1)
                process(1)
                return 0

            lax.fori_loop(0, (n_ch + 1) // 2, pair, 0)
            wait(0)
            pltpu.sync_copy(
                acc_v.at[pl.ds(0, sublen * dh)],
                out_hbm.at[pl.ds(r0 * dh, sublen * dh)])

    return k(m_flat, dstp, rowptr, zeros)


def _tc_pool(zr, batch_r, Nb=2000):
    N, F = zr.shape

    def kern(z_ref, b_ref, o_ref):
        brow = b_ref[...].reshape(1, Nb)
        rows = lax.broadcasted_iota(jnp.int32, (512, Nb), 0)
        oht = (rows == jnp.broadcast_to(brow, (512, Nb))).astype(jnp.float32)
        zcat = jnp.concatenate(
            [z_ref[...], jnp.ones((Nb, 128), jnp.float32)], axis=1)
        part = jnp.dot(oht, zcat, preferred_element_type=jnp.float32)

        @pl.when(pl.program_id(0) == 0)
        def _():
            o_ref[...] = part

        @pl.when(pl.program_id(0) > 0)
        def _():
            o_ref[...] += part

    return pl.pallas_call(
        kern,
        grid=(N // Nb,),
        in_specs=[
            pl.BlockSpec((Nb, F), lambda i: (i, 0)),
            pl.BlockSpec((1, 1, Nb), lambda i: (i, 0, 0)),
        ],
        out_specs=pl.BlockSpec((512, F + 128), lambda i: (0, 0)),
        out_shape=jax.ShapeDtypeStruct((512, F + 128), jnp.float32),
    )(zr, batch_r)


def _stats_block(y, dh):
    s = jnp.sum(y, axis=0, keepdims=True)
    q = jnp.sum(y * y, axis=0, keepdims=True)
    rows = lax.broadcasted_iota(jnp.int32, (8, dh), 0)
    return jnp.where(rows == 0, jnp.broadcast_to(s, (8, dh)),
                     jnp.where(rows == 1, jnp.broadcast_to(q, (8, dh)), 0.0))


def _tc_mm2_stats(Xi, Xj, Wa, Wb, b, Eb):
    E, d_i = Xi.shape
    d_j = Xj.shape[1]
    dh = Wa.shape[1]
    bb = jnp.broadcast_to(b[None, :], (8, dh))

    def kern(xi_ref, xj_ref, wa_ref, wb_ref, b_ref, y_ref, st_ref):
        y = jnp.dot(xi_ref[...], wa_ref[...],
                    preferred_element_type=jnp.float32)
        y = y + jnp.dot(xj_ref[...], wb_ref[...],
                        preferred_element_type=jnp.float32)
        y = y + b_ref[...][0:1, :]
        y_ref[...] = y
        st = _stats_block(y, dh)

        @pl.when(pl.program_id(0) == 0)
        def _():
            st_ref[...] = st

        @pl.when(pl.program_id(0) > 0)
        def _():
            st_ref[...] += st

    return pl.pallas_call(
        kern,
        grid=(E // Eb,),
        in_specs=[
            pl.BlockSpec((Eb, d_i), lambda i: (i, 0)),
            pl.BlockSpec((Eb, d_j), lambda i: (i, 0)),
            pl.BlockSpec((d_i, dh), lambda i: (0, 0)),
            pl.BlockSpec((d_j, dh), lambda i: (0, 0)),
            pl.BlockSpec((8, dh), lambda i: (0, 0)),
        ],
        out_specs=[
            pl.BlockSpec((Eb, dh), lambda i: (i, 0)),
            pl.BlockSpec((8, dh), lambda i: (0, 0)),
        ],
        out_shape=[
            jax.ShapeDtypeStruct((E, dh), jnp.float32),
            jax.ShapeDtypeStruct((8, dh), jnp.float32),
        ],
    )(Xi, Xj, Wa, Wb, bb)


def _tc_affine_mm_stats(Y1, sc, sh, W2, b2, Eb):
    E, dh_in = Y1.shape
    dh = W2.shape[1]
    scb = jnp.broadcast_to(sc[None, :], (8, dh_in))
    shb = jnp.broadcast_to(sh[None, :], (8, dh_in))
    bb = jnp.broadcast_to(b2[None, :], (8, dh))

    def kern(y1_ref, sc_ref, sh_ref, w_ref, b_ref, y_ref, st_ref):
        z = jnp.maximum(y1_ref[...] * sc_ref[...][0:1, :]
                        + sh_ref[...][0:1, :], 0.0)
        y = jnp.dot(z, w_ref[...], preferred_element_type=jnp.float32)
        y = y + b_ref[...][0:1, :]
        y_ref[...] = y
        st = _stats_block(y, dh)

        @pl.when(pl.program_id(0) == 0)
        def _():
            st_ref[...] = st

        @pl.when(pl.program_id(0) > 0)
        def _():
            st_ref[...] += st

    return pl.pallas_call(
        kern,
        grid=(E // Eb,),
        in_specs=[
            pl.BlockSpec((Eb, dh_in), lambda i: (i, 0)),
            pl.BlockSpec((8, dh_in), lambda i: (0, 0)),
            pl.BlockSpec((8, dh_in), lambda i: (0, 0)),
            pl.BlockSpec((dh_in, dh), lambda i: (0, 0)),
            pl.BlockSpec((8, dh), lambda i: (0, 0)),
        ],
        out_specs=[
            pl.BlockSpec((Eb, dh), lambda i: (i, 0)),
            pl.BlockSpec((8, dh), lambda i: (0, 0)),
        ],
        out_shape=[
            jax.ShapeDtypeStruct((E, dh), jnp.float32),
            jax.ShapeDtypeStruct((8, dh), jnp.float32),
        ],
    )(Y1, scb, shb, W2, bb)


def _tc_affine_relu_flat(Y, sc, sh, Eb, epad):
    E, dh = Y.shape
    scb = jnp.broadcast_to(sc[None, :], (8, dh))
    shb = jnp.broadcast_to(sh[None, :], (8, dh))

    def kern(y_ref, sc_ref, sh_ref, o_ref):
        o_ref[...] = jnp.maximum(
            y_ref[...] * sc_ref[...][0:1, :] + sh_ref[...][0:1, :], 0.0)

    return pl.pallas_call(
        kern,
        grid=(E // Eb,),
        in_specs=[
            pl.BlockSpec((Eb, dh), lambda i: (i, 0)),
            pl.BlockSpec((8, dh), lambda i: (0, 0)),
            pl.BlockSpec((8, dh), lambda i: (0, 0)),
        ],
        out_specs=pl.BlockSpec((Eb, dh), lambda i: (i, 0)),
        out_shape=jax.ShapeDtypeStruct((epad, dh), jnp.float32),
    )(Y, scb, shb)


def _tc_affine_relu(Y, sc, sh, Eb):
    E, dh = Y.shape
    scb = jnp.broadcast_to(sc[None, :], (8, dh))
    shb = jnp.broadcast_to(sh[None, :], (8, dh))

    def kern(y_ref, sc_ref, sh_ref, o_ref):
        o_ref[...] = jnp.maximum(
            y_ref[...] * sc_ref[...][0:1, :] + sh_ref[...][0:1, :], 0.0)

    return pl.pallas_call(
        kern,
        grid=(E // Eb,),
        in_specs=[
            pl.BlockSpec((Eb, dh), lambda i: (i, 0)),
            pl.BlockSpec((8, dh), lambda i: (0, 0)),
            pl.BlockSpec((8, dh), lambda i: (0, 0)),
        ],
        out_specs=pl.BlockSpec((Eb, dh), lambda i: (i, 0)),
        out_shape=jax.ShapeDtypeStruct((E, dh), jnp.float32),
    )(Y, scb, shb)


def _tc_combine(part, Nb=1792):
    dh = part.shape[1]

    def kern(p_ref, o_ref):
        v = p_ref[...]
        if dh < 128:
            v = jnp.concatenate(
                [v, jnp.zeros((Nb, 128 - dh), jnp.float32)], axis=1)
        o_ref[...] = v

    return pl.pallas_call(
        kern,
        grid=(NPAD // Nb,),
        in_specs=[pl.BlockSpec((Nb, dh), lambda i: (i, 0))],
        out_specs=pl.BlockSpec((Nb, 128), lambda i: (i, 0)),
        out_shape=jax.ShapeDtypeStruct((NPAD, 128), jnp.float32),
    )(part)


def _tc_head_final(pool, ncol, W2a, W2b, b2, W3, b3):
    Bp = pool.shape[0]
    b2b = jnp.broadcast_to(b2[None, :], (8, 256))
    b3b = jnp.broadcast_to(b3[None, :], (8, 128))
    W2bb = jnp.broadcast_to(W2b[None, :], (8, 256))

    def kern(s_ref, n_ref, w2a_ref, w2b_ref, b2_ref, w3_ref, b3_ref, o_ref):
        sv = s_ref[...]
        sm = sv[:, 0:384]
        cnt = sv[:, 384:385]
        pooled = sm / jnp.maximum(cnt, 1.0)
        ncon = n_ref[...][:, 0:1]
        z = jnp.dot(pooled, w2a_ref[...], preferred_element_type=jnp.float32)
        z = z + ncon * w2b_ref[...][0:1, :]
        z = jnp.maximum(z + b2_ref[...][0:1, :], 0.0)
        o = jnp.dot(z, w3_ref[...], preferred_element_type=jnp.float32)
        o = o + b3_ref[...][0:1, :]
        o_ref[...] = jax.nn.sigmoid(o)

    return pl.pallas_call(
        kern,
        out_shape=jax.ShapeDtypeStruct((Bp, 128), jnp.float32),
    )(pool, ncol, W2a, W2bb, b2b, W3, b3b)


def _affine(stats, n, g, b):
    s = stats[0]
    q = stats[1]
    mean = s / n
    var = q / n - mean * mean
    sc = g * jax.lax.rsqrt(var + EPS)
    sh = b - sc * mean
    return sc, sh


def kernel(x, edge_index, batch, Nconstituents, params):
    src = edge_index[0]
    dst = edge_index[1]
    N = x.shape[0]
    E = src.shape[0]
    B = Nconstituents.shape[0]
    Eb = 3200
    EPAD = E + 2048
    WIN = NPAD // NW

    perm = jnp.argsort(dst)
    dst_s = jnp.asarray(dst[perm], jnp.int32)
    src_s = jnp.asarray(src[perm], jnp.int32)
    bsub = jnp.minimum(jnp.arange(3, dtype=jnp.int32) * 528, WIN)
    bounds = (jnp.arange(32, dtype=jnp.int32)[:, None] * WIN
              + bsub[None, :]).reshape(-1)
    bounds = jnp.concatenate(
        [bounds, jnp.full((1,), NPAD, jnp.int32)])
    rowptr = jnp.pad(
        jnp.searchsorted(dst_s, bounds).astype(jnp.int32), (0, 7))
    bounds1 = jnp.arange(33, dtype=jnp.int32) * WIN
    rowptr1 = jnp.pad(
        jnp.searchsorted(dst_s, bounds1).astype(jnp.int32), (0, 71))
    dstp = jnp.pad(dst_s, (0, EPAD - E), constant_values=2 ** 30)

    h = jnp.pad(x, ((0, NPAD - N), (0, D - x.shape[1])))
    d_true = 3
    xs = []
    for p in params["convs"]:
        W1 = p["W1"]
        W1a = jnp.pad(W1[:d_true], ((0, D - d_true), (0, 0)))
        W1b = jnp.pad(W1[d_true:], ((0, D - d_true), (0, 0)))
        Xi, Xj = _sc_gather(h, dst_s, src_s)
        Y1, st1 = _tc_mm2_stats(Xi, Xj, W1a - W1b, W1b, p["b1"], Eb)
        sc1, sh1 = _affine(st1, E, p["bn1"]["g"], p["bn1"]["b"])
        Y2, st2 = _tc_affine_mm_stats(Y1, sc1, sh1, p["W2"], p["b2"], Eb)
        sc2, sh2 = _affine(st2, E, p["bn2"]["g"], p["bn2"]["b"])
        dh = W1.shape[1]
        m2d = _tc_affine_relu_flat(Y2, sc2, sh2, Eb, EPAD)
        rp = rowptr if dh == 128 else rowptr1
        flat = _sc_scatter(m2d.reshape(EPAD * dh), dstp, rp, dh)
        h = _tc_combine(flat.reshape(NPAD, dh))
        xs.append(h[:N, :W1.shape[1]])
        d_true = W1.shape[1]

    z = jnp.concatenate(xs, axis=1)
    Y, stH = _tc_mm2_stats(z[:, :256], z[:, 256:],
                           params["seq1"]["W"][:256], params["seq1"]["W"][256:],
                           params["seq1"]["b"], 2000)
    scH, shH = _affine(stH, N, params["seq1"]["bn"]["g"],
                       params["seq1"]["bn"]["b"])
    zr = _tc_affine_relu(Y, scH, shH, 2000)

    batch_r = jnp.asarray(batch, jnp.int32).reshape(25, 1, 2000)
    pool = _tc_pool(zr, batch_r)

    Bp = 512
    ncol = jnp.zeros((Bp, 128), jnp.float32).at[:B, 0].set(Nconstituents)
    W2 = params["seq2"]["W"]
    W3 = jnp.pad(params["lin"]["W"], ((0, 0), (0, 127)))
    b3 = jnp.pad(params["lin"]["b"], (0, 127))
    out = _tc_head_final(pool, ncol, W2[:384], W2[384],
                         params["seq2"]["b"], W3, b3)
    return out[:B, 0:1]
